# baseline (speedup 1.0000x reference)
import jax
import jax.numpy as jnp
from jax.experimental import pallas as pl

NUM_LABELS = 1000
MEM_PER_LABEL = 128
MODEL_DIM = 512
ROWS = NUM_LABELS * MEM_PER_LABEL
R_BLOCK = 1000


def _masked_copy_kernel(mem_ref, mask_ref, flag_ref, out_ref):
    valid = mask_ref[...] * flag_ref[...]
    out_ref[...] = mem_ref[...] * valid


def kernel(memory, memory_mask, memory_collected_flag):
    flat = memory.reshape(ROWS, MODEL_DIM)
    mask_col = memory_mask.reshape(ROWS, 1).astype(jnp.float32)
    flag_col = (
        jnp.broadcast_to(memory_collected_flag[:, None], (NUM_LABELS, MEM_PER_LABEL))
        .reshape(ROWS, 1)
        .astype(jnp.float32)
    )
    grid = (ROWS // R_BLOCK,)
    return pl.pallas_call(
        _masked_copy_kernel,
        grid=grid,
        in_specs=[
            pl.BlockSpec((R_BLOCK, MODEL_DIM), lambda i: (i, 0)),
            pl.BlockSpec((R_BLOCK, 1), lambda i: (i, 0)),
            pl.BlockSpec((R_BLOCK, 1), lambda i: (i, 0)),
        ],
        out_specs=pl.BlockSpec((R_BLOCK, MODEL_DIM), lambda i: (i, 0)),
        out_shape=jax.ShapeDtypeStruct((ROWS, MODEL_DIM), jnp.float32),
    )(flat, mask_col, flag_col)


# speedup vs baseline: 1.0482x; 1.0482x over previous
import jax
import jax.numpy as jnp
from jax.experimental import pallas as pl
from jax.experimental.pallas import tpu as pltpu

NUM_LABELS = 1000
MEM_PER_LABEL = 128
MODEL_DIM = 512
ROWS = NUM_LABELS * MEM_PER_LABEL
R_BLOCK = 4000


def _masked_copy_kernel(mem_ref, mask_ref, flag_ref, out_ref):
    valid = mask_ref[...] * flag_ref[...]
    out_ref[...] = mem_ref[...] * valid


def kernel(memory, memory_mask, memory_collected_flag):
    flat = memory.reshape(ROWS, MODEL_DIM)
    mask_col = memory_mask.reshape(ROWS, 1).astype(jnp.float32)
    flag_col = (
        jnp.broadcast_to(memory_collected_flag[:, None], (NUM_LABELS, MEM_PER_LABEL))
        .reshape(ROWS, 1)
        .astype(jnp.float32)
    )
    grid = (ROWS // R_BLOCK,)
    return pl.pallas_call(
        _masked_copy_kernel,
        grid=grid,
        in_specs=[
            pl.BlockSpec((R_BLOCK, MODEL_DIM), lambda i: (i, 0)),
            pl.BlockSpec((R_BLOCK, 1), lambda i: (i, 0)),
            pl.BlockSpec((R_BLOCK, 1), lambda i: (i, 0)),
        ],
        out_specs=pl.BlockSpec((R_BLOCK, MODEL_DIM), lambda i: (i, 0)),
        out_shape=jax.ShapeDtypeStruct((ROWS, MODEL_DIM), jnp.float32),
        compiler_params=pltpu.CompilerParams(
            dimension_semantics=("parallel",),
        ),
    )(flat, mask_col, flag_col)
